# baseline (speedup 1.0000x reference)
import functools

import jax
import jax.numpy as jnp
from jax import lax
from jax.experimental import pallas as pl
from jax.experimental.pallas import tpu as pltpu
from jax.experimental.pallas import tpu_sc as plsc

_NC = 2
_NS = 16
_NW = _NC * _NS

_CH = 256


def _gather_call(seq: int, batch: int, vocab: int, dim: int):
    n_flat = seq * batch
    b_per_w = n_flat // _NW
    n_units = b_per_w // _CH
    units_per_s = batch // _CH
    assert n_units % 2 == 0 and n_units >= 6
    mesh = plsc.VectorSubcoreMesh(
        core_axis_name="c", subcore_axis_name="s", num_cores=_NC, num_subcores=_NS
    )

    out_shape = jax.ShapeDtypeStruct((seq, 8, batch // 128, 8, 128), jnp.float32)

    @functools.partial(
        pl.kernel,
        out_type=out_shape,
        mesh=mesh,
        compiler_params=pltpu.CompilerParams(
            use_tc_tiling_on_sc=False, needs_layout_passes=False
        ),
        scratch_types=[
            pltpu.VMEM((b_per_w,), jnp.int32),
            pltpu.VMEM((2, _CH, 64), jnp.float32),
            pltpu.VMEM((2, 8, _CH // 128, 8, 128), jnp.float32),
            pltpu.SemaphoreType.DMA,
            pltpu.SemaphoreType.DMA,
            pltpu.SemaphoreType.DMA,
            pltpu.SemaphoreType.DMA,
        ],
    )
    def grab(ids_hbm, table_hbm, out_hbm, idx_all, rows_v, tout_v,
             gs0, gs1, ws0, ws1):
        wid = lax.axis_index("s") * _NC + lax.axis_index("c")
        base = wid * b_per_w
        pltpu.sync_copy(ids_hbm.at[pl.ds(base, b_per_w)], idx_all)

        gsem = (gs0, gs1)
        wsem = (ws0, ws1)
        u_base = wid * n_units
        lane = lax.iota(jnp.int32, 16)
        zeros = jnp.zeros((16,), jnp.int32)
        n_cbu = _CH // 128

        def gather(u, b):
            return pltpu.make_async_copy(
                table_hbm.at[idx_all.at[pl.ds(u * _CH, _CH)]], rows_v.at[b],
                gsem[b],
            )

        def wback(u, b):
            gu = u_base + u
            s = gu // units_per_s
            q = gu - s * units_per_s
            return pltpu.make_async_copy(
                tout_v.at[b],
                out_hbm.at[s, :, pl.ds(q * n_cbu, n_cbu), :, :],
                wsem[b],
            )

        def transpose(b):
            def body(gi, _):
                t0 = gi * 16
                cbu = lax.shift_right_logical(gi, 3)
                c0 = lax.bitwise_and(gi, 7) * 16
                tok = lane + t0

                @plsc.parallel_loop(0, 4, unroll=2)
                def dloop(dq):
                    dv = zeros + dq * 16
                    fb0 = dq * 2
                    for k in range(16):
                        x = plsc.load_gather(rows_v.at[b], [tok, dv + k])
                        tout_v[b, fb0 + k // 8, cbu, k % 8, pl.ds(c0, 16)] = x

                return 0

            lax.fori_loop(0, _CH // 16, body, 0)

        gather(0, 0).start()
        gather(1, 1).start()

        def unit(u, b, first, last):
            gather(u, b).wait()
            if not first:
                wback(u - 2, b).wait()
            transpose(b)
            wback(u, b).start()
            if not last:
                gather(u + 2, b).start()

        unit(0, 0, True, False)
        unit(1, 1, True, False)

        def body(j, _):
            u0 = 2 * j
            unit(u0, 0, False, False)
            unit(u0 + 1, 1, False, False)
            return 0

        lax.fori_loop(1, n_units // 2 - 1, body, 0)

        unit(n_units - 2, 0, False, True)
        unit(n_units - 1, 1, False, True)
        wback(n_units - 2, 0).wait()
        wback(n_units - 1, 1).wait()

    return grab


def _pack_call(vocab: int, dim: int):
    n_cb = (vocab + 127) // 128
    mesh = plsc.VectorSubcoreMesh(
        core_axis_name="c", subcore_axis_name="s", num_cores=_NC, num_subcores=_NS
    )
    out_shape = jax.ShapeDtypeStruct((vocab // 2, 2 * dim), jnp.float32)

    @functools.partial(
        pl.kernel,
        out_type=out_shape,
        mesh=mesh,
        compiler_params=pltpu.CompilerParams(
            use_tc_tiling_on_sc=True,
            needs_layout_passes=False,
            disable_bounds_checks=True,
        ),
        scratch_types=[
            pltpu.VMEM((2, dim, 128), jnp.float32),
            pltpu.VMEM((2, 64, 2 * dim), jnp.float32),
            pltpu.SemaphoreType.DMA,
            pltpu.SemaphoreType.DMA,
            pltpu.SemaphoreType.DMA,
            pltpu.SemaphoreType.DMA,
        ],
    )
    def pack(tab_hbm, out_hbm, blk_v, pk_v, gs0, gs1, ws0, ws1):
        wid = lax.axis_index("s") * _NC + lax.axis_index("c")
        gsem = (gs0, gs1)
        wsem = (ws0, ws1)
        lane = lax.iota(jnp.int32, 16)
        zeros = jnp.zeros((16,), jnp.int32)

        def cb_of(k):
            return wid + k * _NW

        def tok0_of(k):
            return pl.multiple_of(cb_of(k) * 128, 128)

        def rd(k, b):
            return pltpu.make_async_copy(
                tab_hbm.at[:, pl.ds(tok0_of(k), 128)], blk_v.at[b], gsem[b]
            )

        def wr(k, b):
            return pltpu.make_async_copy(
                pk_v.at[b],
                out_hbm.at[pl.ds(pl.multiple_of(tok0_of(k) // 2, 64), 64)],
                wsem[b],
            )

        lanes_g = [lane + g * 16 for g in range(dim // 16)]

        def transpose(b):
            @plsc.parallel_loop(0, 64, unroll=4)
            def body(j):
                jv = zeros + 2 * j
                jv1 = jv + 1
                for g in range(dim // 16):
                    x0 = plsc.load_gather(blk_v.at[b], [lanes_g[g], jv])
                    pk_v[b, j, pl.ds(g * 16, 16)] = x0
                    x1 = plsc.load_gather(blk_v.at[b], [lanes_g[g], jv1])
                    pk_v[b, j, pl.ds(dim + g * 16, 16)] = x1

        K = -(-n_cb // _NW)
        assert _NW * (K - 1) <= n_cb <= _NW * K and K >= 4
        assert (K - 1) % 2 == 0

        rd(0, 0).start()
        rd(1, 1).start()

        def unit(k, b):
            rd(k, b).wait()

            @pl.when(k >= 2)
            def _():
                wr(k - 2, b).wait()

            transpose(b)
            wr(k, b).start()

            @pl.when(cb_of(k + 2) < n_cb)
            def _():
                rd(k + 2, b).start()

        def body2(j, _):
            unit(2 * j, 0)
            unit(2 * j + 1, 1)
            return 0

        lax.fori_loop(0, (K - 1) // 2, body2, 0)

        kt = K - 1
        has_tail = cb_of(kt) < n_cb
        part = (vocab % 128) // 2
        is_part = (cb_of(kt) == n_cb - 1) if part else False

        def wr_part(k):
            return pltpu.make_async_copy(
                pk_v.at[0, pl.ds(0, part)],
                out_hbm.at[pl.ds(pl.multiple_of(tok0_of(k) // 2, 64), part)],
                wsem[0],
            )

        @pl.when(has_tail)
        def _():
            rd(kt, 0).wait()
            wr(kt - 2, 0).wait()
            transpose(0)

            if part:
                @pl.when(is_part)
                def _():
                    wr_part(kt).start()
                    wr_part(kt).wait()

                @pl.when(jnp.logical_not(is_part))
                def _():
                    wr(kt, 0).start()
                    wr(kt, 0).wait()
            else:
                wr(kt, 0).start()
                wr(kt, 0).wait()

        @pl.when(jnp.logical_not(has_tail))
        def _():
            wr(kt - 2, 0).wait()

        wr(kt - 1, 1).wait()

    return pack


def kernel(token_ids, table):
    batch, seq = token_ids.shape
    vocab, dim = table.shape
    ids_t = token_ids.astype(jnp.int32).T.reshape(seq * batch)
    table_rm = _pack_call(vocab, dim)(table.T).reshape(vocab, dim)
    out5 = _gather_call(seq, batch, vocab, dim)(ids_t, table_rm)
    return out5.transpose(2, 4, 0, 1, 3).reshape(batch, seq, dim)


# speedup vs baseline: 1.1798x; 1.1798x over previous
import functools

import jax
import jax.numpy as jnp
from jax import lax
from jax.experimental import pallas as pl
from jax.experimental.pallas import tpu as pltpu
from jax.experimental.pallas import tpu_sc as plsc

_NC = 2
_NS = 16
_NW = _NC * _NS

_CH = 256


def _gather_call(seq: int, batch: int, vocab: int, dim: int):
    n_flat = seq * batch
    b_per_w = n_flat // _NW
    n_units = b_per_w // _CH
    units_per_s = batch // _CH
    assert n_units % 2 == 0 and n_units >= 6
    mesh = plsc.VectorSubcoreMesh(
        core_axis_name="c", subcore_axis_name="s", num_cores=_NC, num_subcores=_NS
    )

    out_shape = jax.ShapeDtypeStruct((seq, 8, batch // 128, 8, 128), jnp.float32)

    @functools.partial(
        pl.kernel,
        out_type=out_shape,
        mesh=mesh,
        compiler_params=pltpu.CompilerParams(
            use_tc_tiling_on_sc=False, needs_layout_passes=False
        ),
        scratch_types=[
            pltpu.VMEM((b_per_w,), jnp.int32),
            pltpu.VMEM((2, _CH, 64), jnp.float32),
            pltpu.VMEM((2, 8, _CH // 128, 8, 128), jnp.float32),
            pltpu.SemaphoreType.DMA,
            pltpu.SemaphoreType.DMA,
            pltpu.SemaphoreType.DMA,
            pltpu.SemaphoreType.DMA,
        ],
    )
    def grab(ids_hbm, table_hbm, out_hbm, idx_all, rows_v, tout_v,
             gs0, gs1, ws0, ws1):
        wid = lax.axis_index("s") * _NC + lax.axis_index("c")
        base = wid * b_per_w
        pltpu.sync_copy(ids_hbm.at[pl.ds(base, b_per_w)], idx_all)

        gsem = (gs0, gs1)
        wsem = (ws0, ws1)
        u_base = wid * n_units
        lane = lax.iota(jnp.int32, 16)
        zeros = jnp.zeros((16,), jnp.int32)
        n_cbu = _CH // 128

        def gather(u, b):
            return pltpu.make_async_copy(
                table_hbm.at[idx_all.at[pl.ds(u * _CH, _CH)]], rows_v.at[b],
                gsem[b],
            )

        def wback(u, b):
            gu = u_base + u
            s = gu // units_per_s
            q = gu - s * units_per_s
            return pltpu.make_async_copy(
                tout_v.at[b],
                out_hbm.at[s, :, pl.ds(q * n_cbu, n_cbu), :, :],
                wsem[b],
            )

        def transpose(b):
            def body(gi, _):
                t0 = gi * 16
                cbu = lax.shift_right_logical(gi, 3)
                c0 = lax.bitwise_and(gi, 7) * 16
                tok = lane + t0

                @plsc.parallel_loop(0, 64, unroll=8)
                def dloop(d):
                    x = plsc.load_gather(rows_v.at[b], [tok, zeros + d])
                    fb = lax.shift_right_logical(d, 3)
                    f = lax.bitwise_and(d, 7)
                    tout_v[b, fb, cbu, f, pl.ds(c0, 16)] = x

                return 0

            lax.fori_loop(0, _CH // 16, body, 0)

        gather(0, 0).start()
        gather(1, 1).start()

        def unit(u, b, first, last):
            gather(u, b).wait()
            if not first:
                wback(u - 2, b).wait()
            transpose(b)
            wback(u, b).start()
            if not last:
                gather(u + 2, b).start()

        unit(0, 0, True, False)
        unit(1, 1, True, False)

        def body(j, _):
            u0 = 2 * j
            unit(u0, 0, False, False)
            unit(u0 + 1, 1, False, False)
            return 0

        lax.fori_loop(1, n_units // 2 - 1, body, 0)

        unit(n_units - 2, 0, False, True)
        unit(n_units - 1, 1, False, True)
        wback(n_units - 2, 0).wait()
        wback(n_units - 1, 1).wait()

    return grab


def _pack_call(vocab: int, dim: int):
    n_cb = (vocab + 127) // 128
    mesh = plsc.VectorSubcoreMesh(
        core_axis_name="c", subcore_axis_name="s", num_cores=_NC, num_subcores=_NS
    )
    out_shape = jax.ShapeDtypeStruct((vocab // 2, 2 * dim), jnp.float32)

    @functools.partial(
        pl.kernel,
        out_type=out_shape,
        mesh=mesh,
        compiler_params=pltpu.CompilerParams(
            use_tc_tiling_on_sc=True,
            needs_layout_passes=False,
            disable_bounds_checks=True,
        ),
        scratch_types=[
            pltpu.VMEM((2, dim, 128), jnp.float32),
            pltpu.VMEM((2, 64, 2 * dim), jnp.float32),
            pltpu.SemaphoreType.DMA,
            pltpu.SemaphoreType.DMA,
            pltpu.SemaphoreType.DMA,
            pltpu.SemaphoreType.DMA,
        ],
    )
    def pack(tab_hbm, out_hbm, blk_v, pk_v, gs0, gs1, ws0, ws1):
        wid = lax.axis_index("s") * _NC + lax.axis_index("c")
        gsem = (gs0, gs1)
        wsem = (ws0, ws1)
        lane = lax.iota(jnp.int32, 16)
        zeros = jnp.zeros((16,), jnp.int32)

        def cb_of(k):
            return wid + k * _NW

        def tok0_of(k):
            return pl.multiple_of(cb_of(k) * 128, 128)

        def rd(k, b):
            return pltpu.make_async_copy(
                tab_hbm.at[:, pl.ds(tok0_of(k), 128)], blk_v.at[b], gsem[b]
            )

        def wr(k, b):
            return pltpu.make_async_copy(
                pk_v.at[b],
                out_hbm.at[pl.ds(pl.multiple_of(tok0_of(k) // 2, 64), 64)],
                wsem[b],
            )

        lanes_g = [lane + g * 16 for g in range(dim // 16)]

        def transpose(b):
            @plsc.parallel_loop(0, 64, unroll=4)
            def body(j):
                jv = zeros + 2 * j
                jv1 = jv + 1
                for g in range(dim // 16):
                    x0 = plsc.load_gather(blk_v.at[b], [lanes_g[g], jv])
                    pk_v[b, j, pl.ds(g * 16, 16)] = x0
                    x1 = plsc.load_gather(blk_v.at[b], [lanes_g[g], jv1])
                    pk_v[b, j, pl.ds(dim + g * 16, 16)] = x1

        K = -(-n_cb // _NW)
        assert _NW * (K - 1) <= n_cb <= _NW * K and K >= 4
        assert (K - 1) % 2 == 0

        rd(0, 0).start()
        rd(1, 1).start()

        def unit(k, b):
            rd(k, b).wait()

            @pl.when(k >= 2)
            def _():
                wr(k - 2, b).wait()

            transpose(b)
            wr(k, b).start()

            @pl.when(cb_of(k + 2) < n_cb)
            def _():
                rd(k + 2, b).start()

        def body2(j, _):
            unit(2 * j, 0)
            unit(2 * j + 1, 1)
            return 0

        lax.fori_loop(0, (K - 1) // 2, body2, 0)

        kt = K - 1
        has_tail = cb_of(kt) < n_cb
        part = (vocab % 128) // 2
        is_part = (cb_of(kt) == n_cb - 1) if part else False

        def wr_part(k):
            return pltpu.make_async_copy(
                pk_v.at[0, pl.ds(0, part)],
                out_hbm.at[pl.ds(pl.multiple_of(tok0_of(k) // 2, 64), part)],
                wsem[0],
            )

        @pl.when(has_tail)
        def _():
            rd(kt, 0).wait()
            wr(kt - 2, 0).wait()
            transpose(0)

            if part:
                @pl.when(is_part)
                def _():
                    wr_part(kt).start()
                    wr_part(kt).wait()

                @pl.when(jnp.logical_not(is_part))
                def _():
                    wr(kt, 0).start()
                    wr(kt, 0).wait()
            else:
                wr(kt, 0).start()
                wr(kt, 0).wait()

        @pl.when(jnp.logical_not(has_tail))
        def _():
            wr(kt - 2, 0).wait()

        wr(kt - 1, 1).wait()

    return pack


def kernel(token_ids, table):
    batch, seq = token_ids.shape
    vocab, dim = table.shape
    ids_t = token_ids.astype(jnp.int32).T.reshape(seq * batch)
    table_rm = _pack_call(vocab, dim)(table.T).reshape(vocab, dim)
    out5 = _gather_call(seq, batch, vocab, dim)(ids_t, table_rm)
    return out5.transpose(2, 4, 0, 1, 3).reshape(batch, seq, dim)


# speedup vs baseline: 1.4062x; 1.1918x over previous
import functools

import jax
import jax.numpy as jnp
from jax import lax
from jax.experimental import pallas as pl
from jax.experimental.pallas import tpu as pltpu
from jax.experimental.pallas import tpu_sc as plsc

_NC = 2
_NS = 16
_NW = _NC * _NS

_CHUNK = 512


def _gather_call(n_flat: int, dim: int):
    b_per_w = n_flat // _NW
    n_chunks = b_per_w // _CHUNK
    n_groups = n_chunks // 2
    assert n_chunks % 2 == 0 and n_groups >= 2
    mesh = plsc.VectorSubcoreMesh(
        core_axis_name="c", subcore_axis_name="s", num_cores=_NC, num_subcores=_NS
    )

    @functools.partial(
        pl.kernel,
        out_type=jax.ShapeDtypeStruct((n_flat, dim), jnp.float32),
        mesh=mesh,
        compiler_params=pltpu.CompilerParams(use_tc_tiling_on_sc=False),
        scratch_types=[
            pltpu.VMEM((b_per_w,), jnp.int32),
            pltpu.VMEM((2, _CHUNK, dim), jnp.float32),
            pltpu.SemaphoreType.DMA,
            pltpu.SemaphoreType.DMA,
            pltpu.SemaphoreType.DMA,
            pltpu.SemaphoreType.DMA,
        ],
    )
    def grab(ids_hbm, table_hbm, out_hbm, idx_all, rows_v, gs0, gs1, ws0, ws1):
        wid = lax.axis_index("s") * _NC + lax.axis_index("c")
        base = wid * b_per_w
        pltpu.sync_copy(ids_hbm.at[pl.ds(base, b_per_w)], idx_all)

        gsem = (gs0, gs1)
        wsem = (ws0, ws1)

        def gather(c, b):
            return pltpu.make_async_copy(
                table_hbm.at[idx_all.at[pl.ds(c * _CHUNK, _CHUNK)]],
                rows_v.at[b],
                gsem[b],
            )

        def wback(c, b):
            return pltpu.make_async_copy(
                rows_v.at[b],
                out_hbm.at[pl.ds(base + c * _CHUNK, _CHUNK)],
                wsem[b],
            )

        gather(0, 0).start()
        gather(1, 1).start()
        gather(0, 0).wait()
        wback(0, 0).start()
        wback(0, 0).wait()
        gather(2, 0).start()
        gather(1, 1).wait()
        wback(1, 1).start()

        def body(gi, _):
            c0 = 2 * gi
            c1 = c0 + 1
            wback(c0 - 1, 1).wait()
            gather(c1, 1).start()
            gather(c0, 0).wait()
            wback(c0, 0).start()
            wback(c0, 0).wait()
            gather(c0 + 2, 0).start()
            gather(c1, 1).wait()
            wback(c1, 1).start()
            return 0

        lax.fori_loop(1, n_groups - 1, body, 0)

        cl0 = n_chunks - 2
        cl1 = n_chunks - 1
        wback(cl0 - 1, 1).wait()
        gather(cl1, 1).start()
        gather(cl0, 0).wait()
        wback(cl0, 0).start()
        wback(cl0, 0).wait()
        gather(cl1, 1).wait()
        wback(cl1, 1).start()
        wback(cl1, 1).wait()

    return grab


def kernel(token_ids, table):
    batch, seq = token_ids.shape
    _, dim = table.shape
    n_flat = batch * seq
    flat_ids = token_ids.reshape(n_flat).astype(jnp.int32)
    out = _gather_call(n_flat, dim)(flat_ids, table)
    return out.reshape(batch, seq, dim)
